# baseline (speedup 1.0000x reference)
import numpy as np
import jax
import jax.numpy as jnp
from jax.experimental import pallas as pl
from jax.experimental.pallas import tpu as pltpu

_H = 512
_PERC_T = 0.59593
_KS = (3, 5)
_ORDER = [0, 5, 1, 6, 2, 7, 3, 8, 4, 9]

_GEOM = {
    3: dict(rows=171, l2=2,
            row_plan=[(0, 171, 0), (1, 171, 0), (2, 170, 0)],
            col_plan=[(0, 171, 0), (3, 171, 0), (6, 170, 0)]),
    5: dict(rows=103, l2=1,
            row_plan=[(4, 102, 1), (0, 103, 0), (1, 103, 0),
                      (2, 102, 0), (3, 102, 0)],
            col_plan=[(12, 102, 1), (0, 103, 0), (3, 103, 0),
                      (6, 102, 0), (9, 102, 0)]),
}


def _resize_weights(n_in, n_out):
    x = (np.arange(n_out) + 0.5) * (n_in / n_out) - 0.5
    j = np.arange(n_in)
    w = np.maximum(0.0, 1.0 - np.abs(j[None, :] - x[:, None]))
    return (w / w.sum(1, keepdims=True)).astype(np.float32)


_WH = _resize_weights(2, 128)
_WW = _resize_weights(5, 128)


def _make_basis30():
    b = np.zeros((30, 128, 384), np.float32)
    for c in range(3):
        for p in range(10):
            img = np.outer(_WH[:, p // 5], _WW[:, p % 5])
            b[c * 10 + p, :, c::3] = img
    return b


_BASIS30 = _make_basis30()


def _neighbors(k):
    neigh = []
    for p in range(k * k):
        dy, dx = p // k, p % k
        ns = []
        if dy > 0:
            ns.append(p - k)
        if dy < k - 1:
            ns.append(p + k)
        if dx > 0:
            ns.append(p - 1)
        if dx < k - 1:
            ns.append(p + 1)
        neigh.append(tuple(ns))
    return neigh


def _metrics_for_k(planes, k, rows):
    kk = k * k
    p_cnt = float(rows * rows)
    big = kk + 2
    shp = planes[0].shape
    if len(shp) == 2:
        vmask = jax.lax.broadcasted_iota(jnp.int32, shp, 1) < rows
    else:
        vmask = (jax.lax.broadcasted_iota(jnp.int32, shp, 1) * 128 +
                 jax.lax.broadcasted_iota(jnp.int32, shp, 2)) < rows

    ctr = planes[(k // 2) * k + (k // 2)]
    m = [(jnp.abs(planes[p] - ctr) * 255.0 <= float(k * 8)) & vmask
         for p in range(kk)]
    mf = [jnp.where(m[p], 1.0, 0.0) for p in range(kk)]

    nb = mf[0]
    for p in range(1, kk):
        nb = nb + mf[p]
    nb = jnp.where(vmask, nb, float(kk))

    neigh = _neighbors(k)
    l0 = tuple(jnp.where(m[p], p + 1, big) for p in range(kk))
    bp = [jnp.where(m[p], 0, big) for p in range(kk)]

    def cc_body(_, lab):
        out = []
        for p in range(kk):
            nl = lab[p]
            for q in neigh[p]:
                nl = jnp.minimum(nl, lab[q])
            out.append(jnp.maximum(nl, bp[p]))
        return tuple(out)

    lab = jax.lax.fori_loop(0, kk, cc_body, l0)

    roots = jnp.where(lab[0] == 1, 1.0, 0.0)
    for p in range(1, kk):
        roots = roots + jnp.where(lab[p] == p + 1, 1.0, 0.0)
    acn = jnp.floor(jnp.sum(roots) / p_cnt)

    s_perc = jnp.sum(jnp.where(vmask & (nb / float(kk) >= _PERC_T),
                               1.0, 0.0))
    perc = jnp.floor(s_perc / p_cnt)

    def area_body(j, amax):
        jv = jnp.where(j == 0, big, j)
        cnt = jnp.where(lab[0] == jv, 1.0, 0.0)
        for p in range(1, kk):
            cnt = cnt + jnp.where(lab[p] == jv, 1.0, 0.0)
        return jnp.maximum(amax, cnt)

    amax = jax.lax.fori_loop(0, kk + 1, area_body,
                             jnp.zeros(shp, jnp.float32))
    amax = jnp.where(vmask, amax, 0.0)
    ama = jnp.floor(jnp.sum(amax) / p_cnt)

    def hist_body(v, acc):
        fd_a, m1_a, m2_a = acc
        cnt = jnp.sum(jnp.where(nb == v.astype(jnp.float32), 1.0, 0.0))
        prob = cnt / p_cnt
        r = (v + 1).astype(jnp.float32)
        return (fd_a + prob / r, m1_a + prob * r, m2_a + prob * prob * r)

    fd, m1, m2 = jax.lax.fori_loop(
        0, kk, hist_body, (jnp.float32(0.0), jnp.float32(0.0),
                           jnp.float32(0.0)))
    lac = (m2 - m1 * m1) / (m1 * m1)
    return [acn, perc, ama, lac, fd]


def _fractal_kernel(x_ref, basis_ref, o_ref, t_ref):
    mets = [[] for _ in range(3)]
    for k in _KS:
        g = _GEOM[k]
        rows, l2, kk = g["rows"], g["l2"], k * k

        for dy in range(k):
            s, n, r0 = g["row_plan"][dy]
            a = x_ref[0, s:_H:k, :, :]
            for j2 in range(l2):
                t_ref[dy, :, j2, :] = jnp.zeros((1536, 128), jnp.float32)
            for i in range(12):
                ch = jnp.transpose(a[:, i, :])
                lo = 128 * i
                if rows <= 128:
                    t_ref[dy, lo:lo + 128, 0, r0:r0 + n] = ch
                else:
                    t_ref[dy, lo:lo + 128, 0, r0:] = ch[:, :128 - r0]
                    t_ref[dy, lo:lo + 128, 1, :r0 + n - 128] = \
                        ch[:, 128 - r0:]

        for c in range(3):
            planes = []
            for p in range(kk):
                dy, dx = p // k, p % k
                s2b, nc, j0 = g["col_plan"][dx]
                if l2 == 1:
                    v = t_ref[dy, s2b + c:1536:3 * k, 0, :]
                    zrow = (1, 128)
                else:
                    v = t_ref[dy, s2b + c:1536:3 * k, :, :]
                    zrow = (1, 2, 128)
                parts = []
                if j0:
                    parts.append(jnp.zeros((j0,) + zrow[1:], jnp.float32))
                parts.append(v)
                tail = rows - j0 - nc
                if tail:
                    parts.append(jnp.zeros((tail,) + zrow[1:], jnp.float32))
                planes.append(jnp.concatenate(parts, axis=0)
                              if len(parts) > 1 else v)
            mets[c].extend(_metrics_for_k(planes, k, rows))

    acc = jnp.zeros((128, 384), jnp.float32)
    for c in range(3):
        for p in range(10):
            acc = acc + mets[c][_ORDER[p]] * basis_ref[c * 10 + p, :, :]
    o_ref[0, :, :] = acc


def kernel(inputs):
    b = inputs.shape[0]
    x2 = inputs.reshape(b, _H, 12, 128)
    basis = jnp.asarray(_BASIS30)
    out = pl.pallas_call(
        _fractal_kernel,
        grid=(b,),
        in_specs=[
            pl.BlockSpec((1, _H, 12, 128), lambda i: (i, 0, 0, 0)),
            pl.BlockSpec((30, 128, 384), lambda i: (0, 0, 0)),
        ],
        out_specs=pl.BlockSpec((1, 128, 384), lambda i: (i, 0, 0)),
        out_shape=jax.ShapeDtypeStruct((b, 128, 384), jnp.float32),
        scratch_shapes=[
            pltpu.VMEM((5, 1536, 2, 128), jnp.float32),
        ],
    )(x2, basis)
    return out.reshape(b, 128, 128, 3)


# speedup vs baseline: 1.2305x; 1.2305x over previous
import numpy as np
import jax
import jax.numpy as jnp
from jax.experimental import pallas as pl
from jax.experimental.pallas import tpu as pltpu

_H = 512
_PERC_T = 0.59593
_KS = (3, 5)
_ORDER = [0, 5, 1, 6, 2, 7, 3, 8, 4, 9]

_GEOM = {
    3: dict(rows=171, l2=2,
            row_plan=[(0, 171, 0), (1, 171, 0), (2, 170, 0)],
            col_plan=[(0, 171, 0), (3, 171, 0), (6, 170, 0)]),
    5: dict(rows=103, l2=1,
            row_plan=[(4, 102, 1), (0, 103, 0), (1, 103, 0),
                      (2, 102, 0), (3, 102, 0)],
            col_plan=[(12, 102, 1), (0, 103, 0), (3, 103, 0),
                      (6, 102, 0), (9, 102, 0)]),
}


def _resize_weights(n_in, n_out):
    x = (np.arange(n_out) + 0.5) * (n_in / n_out) - 0.5
    j = np.arange(n_in)
    w = np.maximum(0.0, 1.0 - np.abs(j[None, :] - x[:, None]))
    return (w / w.sum(1, keepdims=True)).astype(np.float32)


_WH = _resize_weights(2, 128)
_WW = _resize_weights(5, 128)


def _make_basis30():
    b = np.zeros((30, 128, 384), np.float32)
    for c in range(3):
        for p in range(10):
            img = np.outer(_WH[:, p // 5], _WW[:, p % 5])
            b[c * 10 + p, :, c::3] = img
    return b


_BASIS30 = _make_basis30()


def _neighbors(k):
    neigh = []
    for p in range(k * k):
        dy, dx = p // k, p % k
        ns = []
        if dy > 0:
            ns.append(p - k)
        if dy < k - 1:
            ns.append(p + k)
        if dx > 0:
            ns.append(p - 1)
        if dx < k - 1:
            ns.append(p + 1)
        neigh.append(tuple(ns))
    return neigh


def _metrics_for_k(planes, k, rows):
    kk = k * k
    p_cnt = float(rows * rows)
    big = kk + 2
    shp = planes[0].shape
    if len(shp) == 2:
        vmask = jax.lax.broadcasted_iota(jnp.int32, shp, 1) < rows
    else:
        vmask = (jax.lax.broadcasted_iota(jnp.int32, shp, 1) * 128 +
                 jax.lax.broadcasted_iota(jnp.int32, shp, 2)) < rows

    ctr = planes[(k // 2) * k + (k // 2)]
    m = [(jnp.abs(planes[p] - ctr) * 255.0 <= float(k * 8)) & vmask
         for p in range(kk)]
    mf = [jnp.where(m[p], 1.0, 0.0) for p in range(kk)]

    nb = mf[0]
    for p in range(1, kk):
        nb = nb + mf[p]
    nb = jnp.where(vmask, nb, float(kk))

    neigh = _neighbors(k)
    l0 = tuple(jnp.where(m[p], p + 1, big) for p in range(kk))
    bp = [jnp.where(m[p], 0, big) for p in range(kk)]

    lab = l0
    for _ in range(kk):
        out = []
        for p in range(kk):
            nl = lab[p]
            for q in neigh[p]:
                nl = jnp.minimum(nl, lab[q])
            out.append(jnp.maximum(nl, bp[p]))
        lab = tuple(out)

    roots = jnp.where(lab[0] == 1, 1.0, 0.0)
    for p in range(1, kk):
        roots = roots + jnp.where(lab[p] == p + 1, 1.0, 0.0)
    acn = jnp.floor(jnp.sum(roots) / p_cnt)

    s_perc = jnp.sum(jnp.where(vmask & (nb / float(kk) >= _PERC_T),
                               1.0, 0.0))
    perc = jnp.floor(s_perc / p_cnt)

    amax = None
    for j in range(kk + 1):
        jv = big if j == 0 else j
        cnt = jnp.where(lab[0] == jv, 1.0, 0.0)
        for p in range(1, kk):
            cnt = cnt + jnp.where(lab[p] == jv, 1.0, 0.0)
        amax = cnt if amax is None else jnp.maximum(amax, cnt)
    amax = jnp.where(vmask, amax, 0.0)
    ama = jnp.floor(jnp.sum(amax) / p_cnt)

    fd = m1 = m2 = jnp.float32(0.0)
    for v in range(kk):
        cnt = jnp.sum(jnp.where(nb == float(v), 1.0, 0.0))
        prob = cnt / p_cnt
        r = float(v + 1)
        fd = fd + prob / r
        m1 = m1 + prob * r
        m2 = m2 + prob * prob * r
    lac = (m2 - m1 * m1) / (m1 * m1)
    return [acn, perc, ama, lac, fd]


def _fractal_kernel(x_ref, basis_ref, o_ref, t_ref):
    mets = [[] for _ in range(3)]
    for k in _KS:
        g = _GEOM[k]
        rows, l2, kk = g["rows"], g["l2"], k * k

        for dy in range(k):
            s, n, r0 = g["row_plan"][dy]
            a = x_ref[0, s:_H:k, :, :]
            for j2 in range(l2):
                t_ref[dy, :, j2, :] = jnp.zeros((1536, 128), jnp.float32)
            for i in range(12):
                ch = jnp.transpose(a[:, i, :])
                lo = 128 * i
                if rows <= 128:
                    t_ref[dy, lo:lo + 128, 0, r0:r0 + n] = ch
                else:
                    t_ref[dy, lo:lo + 128, 0, r0:] = ch[:, :128 - r0]
                    t_ref[dy, lo:lo + 128, 1, :r0 + n - 128] = \
                        ch[:, 128 - r0:]

        for c in range(3):
            planes = []
            for p in range(kk):
                dy, dx = p // k, p % k
                s2b, nc, j0 = g["col_plan"][dx]
                if l2 == 1:
                    v = t_ref[dy, s2b + c:1536:3 * k, 0, :]
                    zrow = (1, 128)
                else:
                    v = t_ref[dy, s2b + c:1536:3 * k, :, :]
                    zrow = (1, 2, 128)
                parts = []
                if j0:
                    parts.append(jnp.zeros((j0,) + zrow[1:], jnp.float32))
                parts.append(v)
                tail = rows - j0 - nc
                if tail:
                    parts.append(jnp.zeros((tail,) + zrow[1:], jnp.float32))
                planes.append(jnp.concatenate(parts, axis=0)
                              if len(parts) > 1 else v)
            mets[c].extend(_metrics_for_k(planes, k, rows))

    acc = jnp.zeros((128, 384), jnp.float32)
    for c in range(3):
        for p in range(10):
            acc = acc + mets[c][_ORDER[p]] * basis_ref[c * 10 + p, :, :]
    o_ref[0, :, :] = acc


def kernel(inputs):
    b = inputs.shape[0]
    x2 = inputs.reshape(b, _H, 12, 128)
    basis = jnp.asarray(_BASIS30)
    out = pl.pallas_call(
        _fractal_kernel,
        grid=(b,),
        in_specs=[
            pl.BlockSpec((1, _H, 12, 128), lambda i: (i, 0, 0, 0)),
            pl.BlockSpec((30, 128, 384), lambda i: (0, 0, 0)),
        ],
        out_specs=pl.BlockSpec((1, 128, 384), lambda i: (i, 0, 0)),
        out_shape=jax.ShapeDtypeStruct((b, 128, 384), jnp.float32),
        scratch_shapes=[
            pltpu.VMEM((5, 1536, 2, 128), jnp.float32),
        ],
    )(x2, basis)
    return out.reshape(b, 128, 128, 3)
